# baseline (speedup 1.0000x reference)
import jax
import jax.numpy as jnp
from jax.experimental import pallas as pl

T = 12
N = 307
C = 64
B = 8
NP = 320
BC = B * C
NW = T - 2
NG = 3


def _body(xt, et, at, wt, bc, out):
    f32 = jnp.float32
    i = pl.program_id(0)
    Xm = xt[i + 1] + et[i + 1]
    Xl = xt[i] + et[i]
    Xr = xt[i + 2] + et[i + 2]

    def glu(y, wtj, bcj):
        parts = []
        for bi in range(B):
            yb = y[bi * C:(bi + 1) * C, :]
            t = jnp.dot(wtj, yb, preferred_element_type=f32) + bcj
            parts.append(t[:C] * jax.nn.sigmoid(t[C:]))
        return jnp.concatenate(parts, axis=0)

    h = None
    acc = None
    for j in range(NG):
        wtj = wt[j]
        bcj = bc[j]
        if j == 0:
            y = (jnp.dot(Xm, at[0:NP, :], preferred_element_type=f32)
                 + jnp.dot(Xl, at[NP:2 * NP, :], preferred_element_type=f32)
                 + jnp.dot(Xr, at[2 * NP:3 * NP, :], preferred_element_type=f32))
        elif j == 1:
            y = jnp.dot(h, at[...], preferred_element_type=f32)
        else:
            y = jnp.dot(h, at[:, 0:NP], preferred_element_type=f32)
        g = glu(y, wtj, bcj)
        if j < NG - 1:
            h = g
            c = g[:, 0:NP]
        else:
            c = g
        acc = c if acc is None else jnp.maximum(acc, c)
    out[0] = acc


def kernel(x, A, temporal_emb, spatial_emb, W, b):
    xt = jnp.transpose(x, (1, 0, 3, 2)).reshape(T, BC, N)
    xt = jnp.pad(xt, ((0, 0), (0, 0), (0, NP - N)))

    te = temporal_emb.reshape(T, C)
    se = spatial_emb.reshape(N, C)
    emb = te[:, :, None] + se.T[None, :, :]
    emb = jnp.pad(emb, ((0, 0), (0, 0), (0, NP - N)))
    embt = jnp.broadcast_to(emb[:, None], (T, B, C, NP)).reshape(T, BC, NP)

    perm = jnp.array([1, 0, 2], dtype=jnp.int32)
    A4 = A.reshape(3, N, 3, N)[perm][:, :, perm]
    Ap = jnp.pad(A4, ((0, 0), (0, NP - N), (0, 0), (0, NP - N)))
    AT = jnp.transpose(Ap.reshape(3 * NP, 3 * NP))

    WT = jnp.transpose(W, (0, 2, 1))
    bcol = b.reshape(NW * NG, 2 * C, 1)

    full = lambda shape: pl.BlockSpec(shape, lambda i: (0,) * len(shape))

    out = pl.pallas_call(
        _body,
        grid=(NW,),
        in_specs=[
            full((T, BC, NP)),
            full((T, BC, NP)),
            full((3 * NP, 3 * NP)),
            pl.BlockSpec((NG, 2 * C, C), lambda i: (i, 0, 0)),
            pl.BlockSpec((NG, 2 * C, 1), lambda i: (i, 0, 0)),
        ],
        out_specs=pl.BlockSpec((1, BC, NP), lambda i: (i, 0, 0)),
        out_shape=jax.ShapeDtypeStruct((NW, BC, NP), jnp.float32),
    )(xt, embt, AT, WT, bcol)

    o = out[:, :, :N].reshape(NW, B, C, N)
    return jnp.transpose(o, (1, 0, 3, 2))


# speedup vs baseline: 1.0841x; 1.0841x over previous
import jax
import jax.numpy as jnp
from jax.experimental import pallas as pl

T = 12
N = 307
C = 64
B = 8
NP = 384
BC = B * C
NW = T - 2
NG = 3


def _body(x0, x1, x2, e0, e1, e2, at, wt, bc, out):
    f32 = jnp.float32
    X0 = x0[0] + e0[0]
    X1 = x1[0] + e1[0]
    X2 = x2[0] + e2[0]

    def glu(y, wtj, bcj):
        parts = []
        for bi in range(B):
            yb = y[bi * C:(bi + 1) * C, :]
            t = jnp.dot(wtj, yb, preferred_element_type=f32) + bcj
            parts.append(t[:C] * jax.nn.sigmoid(t[C:]))
        return jnp.concatenate(parts, axis=0)

    h = None
    acc = None
    for j in range(NG):
        wtj = wt[j]
        bcj = bc[j]
        if j == 0:
            y = (jnp.dot(X0, at[0:NP, :], preferred_element_type=f32)
                 + jnp.dot(X1, at[NP:2 * NP, :], preferred_element_type=f32)
                 + jnp.dot(X2, at[2 * NP:3 * NP, :], preferred_element_type=f32))
        elif j == 1:
            y = jnp.dot(h, at[...], preferred_element_type=f32)
        else:
            y = jnp.dot(h, at[:, NP:2 * NP], preferred_element_type=f32)
        g = glu(y, wtj, bcj)
        if j < NG - 1:
            h = g
            c = g[:, NP:2 * NP]
        else:
            c = g
        acc = c if acc is None else jnp.maximum(acc, c)
    out[0] = acc


def kernel(x, A, temporal_emb, spatial_emb, W, b):
    xt = jnp.transpose(x, (1, 0, 3, 2)).reshape(T, BC, N)
    xt = jnp.pad(xt, ((0, 0), (0, 0), (0, NP - N)))

    te = temporal_emb.reshape(T, C)
    se = spatial_emb.reshape(N, C)
    emb = te[:, :, None] + se.T[None, :, :]
    emb = jnp.pad(emb, ((0, 0), (0, 0), (0, NP - N)))
    embt = jnp.broadcast_to(emb[:, None], (T, B, C, NP)).reshape(T, BC, NP)

    A4 = A.reshape(3, N, 3, N)
    Ap = jnp.pad(A4, ((0, 0), (0, NP - N), (0, 0), (0, NP - N)))
    AT = jnp.transpose(Ap.reshape(3 * NP, 3 * NP))

    WT = jnp.transpose(W, (0, 2, 1))
    bcol = b.reshape(NW * NG, 2 * C, 1)

    blk_x = pl.BlockSpec((1, BC, NP), lambda i: (i, 0, 0))
    blk_x1 = pl.BlockSpec((1, BC, NP), lambda i: (i + 1, 0, 0))
    blk_x2 = pl.BlockSpec((1, BC, NP), lambda i: (i + 2, 0, 0))

    out = pl.pallas_call(
        _body,
        grid=(NW,),
        in_specs=[
            blk_x, blk_x1, blk_x2,
            blk_x, blk_x1, blk_x2,
            pl.BlockSpec((3 * NP, 3 * NP), lambda i: (0, 0)),
            pl.BlockSpec((NG, 2 * C, C), lambda i: (i, 0, 0)),
            pl.BlockSpec((NG, 2 * C, 1), lambda i: (i, 0, 0)),
        ],
        out_specs=pl.BlockSpec((1, BC, NP), lambda i: (i, 0, 0)),
        out_shape=jax.ShapeDtypeStruct((NW, BC, NP), jnp.float32),
    )(xt, xt, xt, embt, embt, embt, AT, WT, bcol)

    o = out[:, :, :N].reshape(NW, B, C, N)
    return jnp.transpose(o, (1, 0, 3, 2))
